# Initial kernel scaffold; baseline (speedup 1.0000x reference)
import jax
import jax.numpy as jnp
from jax.experimental import pallas as pl


def kernel(values, E_u, E_v, edge_index):
    raise NotImplementedError("write your pallas kernel here")


# speedup vs baseline: 3.3700x; 3.3700x over previous
import functools

import jax
import jax.numpy as jnp
from jax import lax
from jax.experimental import pallas as pl
from jax.experimental.pallas import tpu as pltpu
from jax.experimental.pallas import tpu_sc as plsc

_N_USERS = 5000
_N_ITEMS = 5000
_N = _N_USERS + _N_ITEMS
_D = 256
_DQ = 64
_NLAYERS = 4
_NE = 160000
_NS = 16
_NC = 2
_CB = 128
_NCH = 79
_ET = _NCH * _CB
_EP = _ET * _NS
_NP = 10240
_RPT = _NP // _NS
_ROW_CHUNKS = [(0, 128), (128, 128), (256, 128), (384, 128), (512, 128)]


def _rsqrt_newton(x):
    i = lax.bitcast_convert_type(x, jnp.int32)
    i = jnp.int32(0x5F3759DF) - lax.shift_right_logical(i, 1)
    y = lax.bitcast_convert_type(i, jnp.float32)
    for _ in range(3):
        y = y * (jnp.float32(1.5) - jnp.float32(0.5) * x * y * y)
    return jnp.where(x > 0, y, jnp.float32(0.0))


def _sc_body(*refs):
    (eq0, eq1, eq2, eq3, row3, col3, val3) = refs[:7]
    lay = refs[7:23]
    fin = refs[23:27]
    (row_t, col_t, w_t, dis_t, gbuf, zbuf, dtmp, sh_acc, sh_deg) = refs[27:]
    eq = [eq0, eq1, eq2, eq3]
    c = lax.axis_index("c")
    s = lax.axis_index("s")

    pltpu.sync_copy(row3.at[s], row_t)
    pltpu.sync_copy(col3.at[s], col_t)
    pltpu.sync_copy(val3.at[s], w_t)

    zv = jnp.zeros((16,), jnp.float32)
    for k in range(40):
        dtmp[pl.ds(16 * k, 16)] = zv

    def _zero_zbuf(i, _):
        for v in range(_DQ // 16):
            zbuf[i, pl.ds(16 * v, 16)] = zv
        return 0
    lax.fori_loop(0, _CB, _zero_zbuf, 0)

    pltpu.sync_copy(dtmp, sh_deg.at[pl.ds(s * 640, 640)])
    plsc.subcore_barrier()

    def _deg_chunk(j, _):
        pltpu.sync_copy(w_t.at[j], sh_deg.at[row_t.at[j]], add=True)
        return 0
    lax.fori_loop(0, _NCH, _deg_chunk, 0)
    plsc.subcore_barrier()

    pltpu.sync_copy(sh_deg.at[pl.ds(s * 640, 640)], dtmp)

    def _dis_vec(k, _):
        x = dtmp[pl.ds(16 * k, 16)]
        dtmp[pl.ds(16 * k, 16)] = _rsqrt_newton(x)
        return 0
    lax.fori_loop(0, 40, _dis_vec, 0)
    pltpu.sync_copy(dtmp, sh_deg.at[pl.ds(s * 640, 640)])
    plsc.subcore_barrier()
    pltpu.sync_copy(sh_deg, dis_t)

    def _wchunk(j, _):
        for i in range(8):
            sl = pl.ds(16 * i, 16)
            r16 = row_t[j, sl]
            c16 = col_t[j, sl]
            v16 = w_t[j, sl]
            a = plsc.load_gather(dis_t, [r16])
            b = plsc.load_gather(dis_t, [c16])
            w_t[j, sl] = a * v16 * b
        return 0
    lax.fori_loop(0, _NCH, _wchunk, 0)

    def _run(qq):
        srcs = [eq[qq]] + [lay[4 * l + qq] for l in range(_NLAYERS)]
        for layer in range(_NLAYERS):
            for off, sz in _ROW_CHUNKS:
                pltpu.sync_copy(zbuf.at[pl.ds(0, sz)],
                                sh_acc.at[pl.ds(s * _RPT + off, sz)])
            plsc.subcore_barrier()

            src = srcs[layer]
            dst = lay[4 * layer + qq]

            def _edge_chunk(j, _):
                pltpu.sync_copy(src.at[col_t.at[j]], gbuf)

                def _scale(i, _):
                    wv = plsc.load_gather(
                        w_t, [jnp.full((16,), j, jnp.int32),
                              jnp.full((16,), i, jnp.int32)])
                    for v in range(_DQ // 16):
                        sl = pl.ds(16 * v, 16)
                        gbuf[i, sl] = gbuf[i, sl] * wv
                    return 0
                lax.fori_loop(0, _CB, _scale, 0)
                pltpu.sync_copy(gbuf, sh_acc.at[row_t.at[j]], add=True)
                return 0
            lax.fori_loop(0, _NCH, _edge_chunk, 0)
            plsc.subcore_barrier()

            for off, sz in _ROW_CHUNKS:
                pltpu.sync_copy(sh_acc.at[pl.ds(s * _RPT + off, sz)],
                                gbuf.at[pl.ds(0, sz)])
                pltpu.sync_copy(gbuf.at[pl.ds(0, sz)],
                                dst.at[pl.ds(s * _RPT + off, sz)])
            plsc.subcore_barrier()

        fifth = jnp.float32(0.2)
        for off, sz in _ROW_CHUNKS:
            nb = s * _RPT + off
            pltpu.sync_copy(srcs[0].at[pl.ds(nb, sz)], zbuf.at[pl.ds(0, sz)])
            for t in range(1, 5):
                pltpu.sync_copy(srcs[t].at[pl.ds(nb, sz)],
                                gbuf.at[pl.ds(0, sz)])

                def _acc(i, _):
                    for v in range(_DQ // 16):
                        sl = pl.ds(16 * v, 16)
                        zbuf[i, sl] = zbuf[i, sl] + gbuf[i, sl]
                    return 0
                lax.fori_loop(0, sz, _acc, 0)

            def _fin(i, _):
                for v in range(_DQ // 16):
                    sl = pl.ds(16 * v, 16)
                    zbuf[i, sl] = zbuf[i, sl] * fifth
                return 0
            lax.fori_loop(0, sz, _fin, 0)
            pltpu.sync_copy(zbuf.at[pl.ds(0, sz)], fin[qq].at[pl.ds(nb, sz)])
        lax.fori_loop(0, _CB, _zero_zbuf, 0)

    @pl.when(c == 0)
    def _():
        _run(0)
        _run(1)

    @pl.when(c == 1)
    def _():
        _run(2)
        _run(3)


_quarter = jax.ShapeDtypeStruct((_NP, _DQ), jnp.float32)

_sc_kernel = functools.partial(
    pl.kernel,
    out_type=[_quarter] * 20,
    mesh=plsc.VectorSubcoreMesh(core_axis_name="c", subcore_axis_name="s",
                                num_cores=_NC, num_subcores=_NS),
    compiler_params=pltpu.CompilerParams(needs_layout_passes=False,
                                         use_tc_tiling_on_sc=False),
    scratch_types=[
        pltpu.VMEM((_NCH, _CB), jnp.int32),
        pltpu.VMEM((_NCH, _CB), jnp.int32),
        pltpu.VMEM((_NCH, _CB), jnp.float32),
        pltpu.VMEM((_NP,), jnp.float32),
        pltpu.VMEM((_CB, _DQ), jnp.float32),
        pltpu.VMEM((_CB, _DQ), jnp.float32),
        pltpu.VMEM((640,), jnp.float32),
        pltpu.VMEM_SHARED((_NP, _DQ), jnp.float32),
        pltpu.VMEM_SHARED((_NP,), jnp.float32),
    ],
)(_sc_body)


def kernel(values, E_u, E_v, edge_index):
    row = edge_index[0].astype(jnp.int32)
    col = edge_index[1].astype(jnp.int32)
    pad = _EP - _NE
    row3 = jnp.concatenate([row, jnp.zeros((pad,), jnp.int32)]).reshape(
        _NS, _NCH, _CB)
    col3 = jnp.concatenate([col, jnp.zeros((pad,), jnp.int32)]).reshape(
        _NS, _NCH, _CB)
    val3 = jnp.concatenate([values, jnp.zeros((pad,), values.dtype)]).reshape(
        _NS, _NCH, _CB)
    E_full = jnp.concatenate(
        [E_u, E_v, jnp.zeros((_NP - _N, _D), jnp.float32)], axis=0)
    eqs = [E_full[:, q * _DQ:(q + 1) * _DQ] for q in range(4)]
    outs = _sc_kernel(*eqs, row3, col3, val3)
    fin = outs[16:20]
    E_final = jnp.concatenate(fin, axis=1)
    return (E_final[:_N_USERS], E_final[_N_USERS:_N])


# speedup vs baseline: 4.0627x; 1.2055x over previous
import functools

import jax
import jax.numpy as jnp
from jax import lax
from jax.experimental import pallas as pl
from jax.experimental.pallas import tpu as pltpu
from jax.experimental.pallas import tpu_sc as plsc

_N_USERS = 5000
_N_ITEMS = 5000
_N = _N_USERS + _N_ITEMS
_D = 256
_DQ = 64
_NLAYERS = 4
_NE = 160000
_NS = 16
_NC = 2
_CB = 128
_NCH = 80
_ET = _NCH * _CB
_EP = _ET * _NS
_NP = 10240
_RPT = _NP // _NS
_ROW_CHUNKS = [(0, 128), (128, 128), (256, 128), (384, 128), (512, 128)]
_NBUF = 4


def _rsqrt_newton(x):
    i = lax.bitcast_convert_type(x, jnp.int32)
    i = jnp.int32(0x5F3759DF) - lax.shift_right_logical(i, 1)
    y = lax.bitcast_convert_type(i, jnp.float32)
    for _ in range(3):
        y = y * (jnp.float32(1.5) - jnp.float32(0.5) * x * y * y)
    return jnp.where(x > 0, y, jnp.float32(0.0))


def _sc_body(*refs):
    (eq0, eq1, eq2, eq3, row3, col3, val3) = refs[:7]
    lay = refs[7:23]
    fin = refs[23:27]
    (row_t, col_t, w_t, dis_t, gbuf0, gbuf1, gbuf2, gbuf3, zbuf, dtmp,
     sg0, sg1, sg2, sg3, ss0, ss1, ss2, ss3, sh_acc, sh_deg) = refs[7 + 20:]
    gbufs = [gbuf0, gbuf1, gbuf2, gbuf3]
    sem_g = [sg0, sg1, sg2, sg3]
    sem_s = [ss0, ss1, ss2, ss3]
    eq = [eq0, eq1, eq2, eq3]
    c = lax.axis_index("c")
    s = lax.axis_index("s")

    pltpu.sync_copy(row3.at[s], row_t)
    pltpu.sync_copy(col3.at[s], col_t)
    pltpu.sync_copy(val3.at[s], w_t)

    zv = jnp.zeros((16,), jnp.float32)
    for k in range(40):
        dtmp[pl.ds(16 * k, 16)] = zv

    def _zero_zbuf(i, _):
        for v in range(_DQ // 16):
            zbuf[i, pl.ds(16 * v, 16)] = zv
        return 0
    lax.fori_loop(0, _CB, _zero_zbuf, 0)

    pltpu.sync_copy(dtmp, sh_deg.at[pl.ds(s * 640, 640)])
    plsc.subcore_barrier()

    def _deg_chunk(j, _):
        pltpu.sync_copy(w_t.at[j], sh_deg.at[row_t.at[j]], add=True)
        return 0
    lax.fori_loop(0, _NCH, _deg_chunk, 0)
    plsc.subcore_barrier()

    pltpu.sync_copy(sh_deg.at[pl.ds(s * 640, 640)], dtmp)

    def _dis_vec(k, _):
        x = dtmp[pl.ds(16 * k, 16)]
        dtmp[pl.ds(16 * k, 16)] = _rsqrt_newton(x)
        return 0
    lax.fori_loop(0, 40, _dis_vec, 0)
    pltpu.sync_copy(dtmp, sh_deg.at[pl.ds(s * 640, 640)])
    plsc.subcore_barrier()
    pltpu.sync_copy(sh_deg, dis_t)

    def _wchunk(j, _):
        for i in range(8):
            sl = pl.ds(16 * i, 16)
            r16 = row_t[j, sl]
            c16 = col_t[j, sl]
            v16 = w_t[j, sl]
            a = plsc.load_gather(dis_t, [r16])
            b = plsc.load_gather(dis_t, [c16])
            w_t[j, sl] = a * v16 * b
        return 0
    lax.fori_loop(0, _NCH, _wchunk, 0)

    def _run(qq):
        srcs = [eq[qq]] + [lay[4 * l + qq] for l in range(_NLAYERS)]
        for layer in range(_NLAYERS):
            for off, sz in _ROW_CHUNKS:
                pltpu.sync_copy(zbuf.at[pl.ds(0, sz)],
                                sh_acc.at[pl.ds(s * _RPT + off, sz)])
            plsc.subcore_barrier()

            src = srcs[layer]
            dst = lay[4 * layer + qq]

            pltpu.async_copy(src.at[col_t.at[0]], gbufs[0], sem_g[0])

            def _edge_quad(jj, _):
                for b in range(_NBUF):
                    j = _NBUF * jj + b
                    bn = (b + 1) % _NBUF
                    pltpu.make_async_copy(
                        src.at[col_t.at[j]], gbufs[b], sem_g[b]).wait()

                    @pl.when(j >= _NBUF - 1)
                    def _():
                        pltpu.make_async_copy(
                            gbufs[bn], sh_acc.at[row_t.at[j]],
                            sem_s[bn]).wait()

                    @pl.when(j + 1 < _NCH)
                    def _():
                        pltpu.async_copy(src.at[col_t.at[j + 1]],
                                         gbufs[bn], sem_g[bn])

                    def _scale(i, _, b=b):
                        wv = plsc.load_gather(
                            w_t, [jnp.full((16,), j, jnp.int32),
                                  jnp.full((16,), i, jnp.int32)])
                        for v in range(_DQ // 16):
                            sl = pl.ds(16 * v, 16)
                            gbufs[b][i, sl] = gbufs[b][i, sl] * wv
                        return 0
                    lax.fori_loop(0, _CB, _scale, 0)
                    pltpu.async_copy(gbufs[b], sh_acc.at[row_t.at[j]],
                                     sem_s[b], add=True)
                return 0
            lax.fori_loop(0, _NCH // _NBUF, _edge_quad, 0)
            for b in range(1, _NBUF):
                pltpu.make_async_copy(
                    gbufs[b], sh_acc.at[row_t.at[_NCH - _NBUF + b]],
                    sem_s[b]).wait()
            plsc.subcore_barrier()

            for off, sz in _ROW_CHUNKS:
                pltpu.sync_copy(sh_acc.at[pl.ds(s * _RPT + off, sz)],
                                gbuf0.at[pl.ds(0, sz)])
                pltpu.sync_copy(gbuf0.at[pl.ds(0, sz)],
                                dst.at[pl.ds(s * _RPT + off, sz)])
            plsc.subcore_barrier()

        fifth = jnp.float32(0.2)
        for off, sz in _ROW_CHUNKS:
            nb = s * _RPT + off
            pltpu.sync_copy(srcs[0].at[pl.ds(nb, sz)], zbuf.at[pl.ds(0, sz)])
            for t in range(1, 5):
                pltpu.sync_copy(srcs[t].at[pl.ds(nb, sz)],
                                gbuf0.at[pl.ds(0, sz)])

                def _acc(i, _):
                    for v in range(_DQ // 16):
                        sl = pl.ds(16 * v, 16)
                        zbuf[i, sl] = zbuf[i, sl] + gbuf0[i, sl]
                    return 0
                lax.fori_loop(0, sz, _acc, 0)

            def _fin(i, _):
                for v in range(_DQ // 16):
                    sl = pl.ds(16 * v, 16)
                    zbuf[i, sl] = zbuf[i, sl] * fifth
                return 0
            lax.fori_loop(0, sz, _fin, 0)
            pltpu.sync_copy(zbuf.at[pl.ds(0, sz)], fin[qq].at[pl.ds(nb, sz)])
        lax.fori_loop(0, _CB, _zero_zbuf, 0)

    @pl.when(c == 0)
    def _():
        _run(0)
        _run(1)

    @pl.when(c == 1)
    def _():
        _run(2)
        _run(3)


_quarter = jax.ShapeDtypeStruct((_NP, _DQ), jnp.float32)

_sc_kernel = functools.partial(
    pl.kernel,
    out_type=[_quarter] * 20,
    mesh=plsc.VectorSubcoreMesh(core_axis_name="c", subcore_axis_name="s",
                                num_cores=_NC, num_subcores=_NS),
    compiler_params=pltpu.CompilerParams(needs_layout_passes=False,
                                         use_tc_tiling_on_sc=False),
    scratch_types=[
        pltpu.VMEM((_NCH, _CB), jnp.int32),
        pltpu.VMEM((_NCH, _CB), jnp.int32),
        pltpu.VMEM((_NCH, _CB), jnp.float32),
        pltpu.VMEM((_NP,), jnp.float32),
        pltpu.VMEM((_CB, _DQ), jnp.float32),
        pltpu.VMEM((_CB, _DQ), jnp.float32),
        pltpu.VMEM((_CB, _DQ), jnp.float32),
        pltpu.VMEM((_CB, _DQ), jnp.float32),
        pltpu.VMEM((_CB, _DQ), jnp.float32),
        pltpu.VMEM((640,), jnp.float32),
        pltpu.SemaphoreType.DMA,
        pltpu.SemaphoreType.DMA,
        pltpu.SemaphoreType.DMA,
        pltpu.SemaphoreType.DMA,
        pltpu.SemaphoreType.DMA,
        pltpu.SemaphoreType.DMA,
        pltpu.SemaphoreType.DMA,
        pltpu.SemaphoreType.DMA,
        pltpu.VMEM_SHARED((_NP, _DQ), jnp.float32),
        pltpu.VMEM_SHARED((_NP,), jnp.float32),
    ],
)(_sc_body)


def kernel(values, E_u, E_v, edge_index):
    row = edge_index[0].astype(jnp.int32)
    col = edge_index[1].astype(jnp.int32)
    pad = _EP - _NE
    row3 = jnp.concatenate([row, jnp.zeros((pad,), jnp.int32)]).reshape(
        _NS, _NCH, _CB)
    col3 = jnp.concatenate([col, jnp.zeros((pad,), jnp.int32)]).reshape(
        _NS, _NCH, _CB)
    val3 = jnp.concatenate([values, jnp.zeros((pad,), values.dtype)]).reshape(
        _NS, _NCH, _CB)
    E_full = jnp.concatenate(
        [E_u, E_v, jnp.zeros((_NP - _N, _D), jnp.float32)], axis=0)
    eqs = [E_full[:, q * _DQ:(q + 1) * _DQ] for q in range(4)]
    outs = _sc_kernel(*eqs, row3, col3, val3)
    fin = outs[16:20]
    E_final = jnp.concatenate(fin, axis=1)
    return (E_final[:_N_USERS], E_final[_N_USERS:_N])


# speedup vs baseline: 4.1171x; 1.0134x over previous
import functools

import jax
import jax.numpy as jnp
from jax import lax
from jax.experimental import pallas as pl
from jax.experimental.pallas import tpu as pltpu
from jax.experimental.pallas import tpu_sc as plsc

_N_USERS = 5000
_N_ITEMS = 5000
_N = _N_USERS + _N_ITEMS
_D = 256
_DQ = 64
_NLAYERS = 4
_NE = 160000
_NS = 16
_NC = 2
_CB = 128
_NCH = 80
_ET = _NCH * _CB
_EP = _ET * _NS
_NP = 10240
_RPT = _NP // _NS
_ROW_CHUNKS = [(0, 128), (128, 128), (256, 128), (384, 128), (512, 128)]
_NBUF = 4


def _rsqrt_newton(x):
    i = lax.bitcast_convert_type(x, jnp.int32)
    i = jnp.int32(0x5F3759DF) - lax.shift_right_logical(i, 1)
    y = lax.bitcast_convert_type(i, jnp.float32)
    for _ in range(3):
        y = y * (jnp.float32(1.5) - jnp.float32(0.5) * x * y * y)
    return jnp.where(x > 0, y, jnp.float32(0.0))


def _sc_body(*refs):
    (eq0, eq1, eq2, eq3, row3, col3, val3) = refs[:7]
    lay = refs[7:23]
    fin = refs[23:27]
    (row_t, col_t, w_t, dis_t, gbuf0, gbuf1, gbuf2, gbuf3, zbuf, dtmp,
     sg0, sg1, sg2, sg3, ss0, ss1, ss2, ss3, sh_acc, sh_deg) = refs[7 + 20:]
    gbufs = [gbuf0, gbuf1, gbuf2, gbuf3]
    sem_g = [sg0, sg1, sg2, sg3]
    sem_s = [ss0, ss1, ss2, ss3]
    eq = [eq0, eq1, eq2, eq3]
    c = lax.axis_index("c")
    s = lax.axis_index("s")

    pltpu.sync_copy(row3.at[s], row_t)
    pltpu.sync_copy(col3.at[s], col_t)
    pltpu.sync_copy(val3.at[s], w_t)

    zv = jnp.zeros((16,), jnp.float32)
    for k in range(40):
        dtmp[pl.ds(16 * k, 16)] = zv

    def _zero_zbuf(i, _):
        for v in range(_DQ // 16):
            zbuf[i, pl.ds(16 * v, 16)] = zv
        return 0
    lax.fori_loop(0, _CB, _zero_zbuf, 0)

    pltpu.sync_copy(dtmp, sh_deg.at[pl.ds(s * 640, 640)])
    plsc.subcore_barrier()

    def _deg_fire(j, _):
        pltpu.async_copy(w_t.at[j], sh_deg.at[row_t.at[j]], ss0, add=True)
        return 0
    lax.fori_loop(0, _NCH, _deg_fire, 0)

    def _deg_drain(j, _):
        pltpu.make_async_copy(w_t.at[j], sh_deg.at[row_t.at[j]], ss0).wait()
        return 0
    lax.fori_loop(0, _NCH, _deg_drain, 0)
    plsc.subcore_barrier()

    pltpu.sync_copy(sh_deg.at[pl.ds(s * 640, 640)], dtmp)

    def _dis_vec(k, _):
        x = dtmp[pl.ds(16 * k, 16)]
        dtmp[pl.ds(16 * k, 16)] = _rsqrt_newton(x)
        return 0
    lax.fori_loop(0, 40, _dis_vec, 0)
    pltpu.sync_copy(dtmp, sh_deg.at[pl.ds(s * 640, 640)])
    plsc.subcore_barrier()
    pltpu.sync_copy(sh_deg, dis_t)

    def _wchunk(j, _):
        for i in range(8):
            sl = pl.ds(16 * i, 16)
            r16 = row_t[j, sl]
            c16 = col_t[j, sl]
            v16 = w_t[j, sl]
            a = plsc.load_gather(dis_t, [r16])
            b = plsc.load_gather(dis_t, [c16])
            w_t[j, sl] = a * v16 * b
        return 0
    lax.fori_loop(0, _NCH, _wchunk, 0)

    def _run(qq):
        srcs = [eq[qq]] + [lay[4 * l + qq] for l in range(_NLAYERS)]
        for off, sz in _ROW_CHUNKS:
            pltpu.sync_copy(zbuf.at[pl.ds(0, sz)],
                            sh_acc.at[pl.ds(s * _RPT + off, sz)])
        plsc.subcore_barrier()
        for layer in range(_NLAYERS):
            src = srcs[layer]
            dst = lay[4 * layer + qq]

            pltpu.async_copy(src.at[col_t.at[0]], gbufs[0], sem_g[0])

            def _edge_quad(jj, _):
                for b in range(_NBUF):
                    j = _NBUF * jj + b
                    bn = (b + 1) % _NBUF
                    pltpu.make_async_copy(
                        src.at[col_t.at[j]], gbufs[b], sem_g[b]).wait()

                    @pl.when(j >= _NBUF - 1)
                    def _():
                        pltpu.make_async_copy(
                            gbufs[bn], sh_acc.at[row_t.at[j]],
                            sem_s[bn]).wait()

                    @pl.when(j + 1 < _NCH)
                    def _():
                        pltpu.async_copy(src.at[col_t.at[j + 1]],
                                         gbufs[bn], sem_g[bn])

                    def _scale(i, _, b=b):
                        wv = plsc.load_gather(
                            w_t, [jnp.full((16,), j, jnp.int32),
                                  jnp.full((16,), i, jnp.int32)])
                        for v in range(_DQ // 16):
                            sl = pl.ds(16 * v, 16)
                            gbufs[b][i, sl] = gbufs[b][i, sl] * wv
                        return 0
                    lax.fori_loop(0, _CB, _scale, 0)
                    pltpu.async_copy(gbufs[b], sh_acc.at[row_t.at[j]],
                                     sem_s[b], add=True)
                return 0
            lax.fori_loop(0, _NCH // _NBUF, _edge_quad, 0)
            for b in range(1, _NBUF):
                pltpu.make_async_copy(
                    gbufs[b], sh_acc.at[row_t.at[_NCH - _NBUF + b]],
                    sem_s[b]).wait()
            plsc.subcore_barrier()

            if layer < _NLAYERS - 1:
                for k, (off, sz) in enumerate(_ROW_CHUNKS):
                    kb = k % _NBUF
                    if k >= _NBUF:
                        pltpu.make_async_copy(
                            gbufs[kb],
                            dst.at[pl.ds(s * _RPT + _ROW_CHUNKS[k - _NBUF][0],
                                         sz)],
                            sem_s[kb]).wait()
                    pltpu.sync_copy(sh_acc.at[pl.ds(s * _RPT + off, sz)],
                                    gbufs[kb])
                    pltpu.sync_copy(zbuf,
                                    sh_acc.at[pl.ds(s * _RPT + off, sz)])
                    pltpu.async_copy(gbufs[kb],
                                     dst.at[pl.ds(s * _RPT + off, sz)],
                                     sem_s[kb])
                nch = len(_ROW_CHUNKS)
                for k in range(max(0, nch - _NBUF), nch):
                    kb = k % _NBUF
                    pltpu.make_async_copy(
                        gbufs[kb],
                        dst.at[pl.ds(s * _RPT + _ROW_CHUNKS[k][0], 128)],
                        sem_s[kb]).wait()
                plsc.subcore_barrier()

        fifth = jnp.float32(0.2)
        for off, sz in _ROW_CHUNKS:
            nb = s * _RPT + off
            pltpu.sync_copy(sh_acc.at[pl.ds(nb, sz)], zbuf)
            for t in range(4):
                pltpu.sync_copy(srcs[t].at[pl.ds(nb, sz)], gbuf0)

                def _acc(i, _):
                    for v in range(_DQ // 16):
                        sl = pl.ds(16 * v, 16)
                        zbuf[i, sl] = zbuf[i, sl] + gbuf0[i, sl]
                    return 0
                lax.fori_loop(0, sz, _acc, 0)

            def _fin(i, _):
                for v in range(_DQ // 16):
                    sl = pl.ds(16 * v, 16)
                    zbuf[i, sl] = zbuf[i, sl] * fifth
                return 0
            lax.fori_loop(0, sz, _fin, 0)
            pltpu.sync_copy(zbuf, fin[qq].at[pl.ds(nb, sz)])
        lax.fori_loop(0, _CB, _zero_zbuf, 0)

    @pl.when(c == 0)
    def _():
        _run(0)
        _run(1)

    @pl.when(c == 1)
    def _():
        _run(2)
        _run(3)


_quarter = jax.ShapeDtypeStruct((_NP, _DQ), jnp.float32)

_sc_kernel = functools.partial(
    pl.kernel,
    out_type=[_quarter] * 20,
    mesh=plsc.VectorSubcoreMesh(core_axis_name="c", subcore_axis_name="s",
                                num_cores=_NC, num_subcores=_NS),
    compiler_params=pltpu.CompilerParams(needs_layout_passes=False,
                                         use_tc_tiling_on_sc=False),
    scratch_types=[
        pltpu.VMEM((_NCH, _CB), jnp.int32),
        pltpu.VMEM((_NCH, _CB), jnp.int32),
        pltpu.VMEM((_NCH, _CB), jnp.float32),
        pltpu.VMEM((_NP,), jnp.float32),
        pltpu.VMEM((_CB, _DQ), jnp.float32),
        pltpu.VMEM((_CB, _DQ), jnp.float32),
        pltpu.VMEM((_CB, _DQ), jnp.float32),
        pltpu.VMEM((_CB, _DQ), jnp.float32),
        pltpu.VMEM((_CB, _DQ), jnp.float32),
        pltpu.VMEM((640,), jnp.float32),
        pltpu.SemaphoreType.DMA,
        pltpu.SemaphoreType.DMA,
        pltpu.SemaphoreType.DMA,
        pltpu.SemaphoreType.DMA,
        pltpu.SemaphoreType.DMA,
        pltpu.SemaphoreType.DMA,
        pltpu.SemaphoreType.DMA,
        pltpu.SemaphoreType.DMA,
        pltpu.VMEM_SHARED((_NP, _DQ), jnp.float32),
        pltpu.VMEM_SHARED((_NP,), jnp.float32),
    ],
)(_sc_body)


def kernel(values, E_u, E_v, edge_index):
    row = edge_index[0].astype(jnp.int32)
    col = edge_index[1].astype(jnp.int32)
    pad = _EP - _NE
    row3 = jnp.concatenate([row, jnp.zeros((pad,), jnp.int32)]).reshape(
        _NS, _NCH, _CB)
    col3 = jnp.concatenate([col, jnp.zeros((pad,), jnp.int32)]).reshape(
        _NS, _NCH, _CB)
    val3 = jnp.concatenate([values, jnp.zeros((pad,), values.dtype)]).reshape(
        _NS, _NCH, _CB)
    E_full = jnp.concatenate(
        [E_u, E_v, jnp.zeros((_NP - _N, _D), jnp.float32)], axis=0)
    eqs = [E_full[:, q * _DQ:(q + 1) * _DQ] for q in range(4)]
    outs = _sc_kernel(*eqs, row3, col3, val3)
    fin = outs[16:20]
    E_final = jnp.concatenate(fin, axis=1)
    return (E_final[:_N_USERS], E_final[_N_USERS:_N])


# speedup vs baseline: 4.4998x; 1.0930x over previous
import functools

import jax
import jax.numpy as jnp
from jax import lax
from jax.experimental import pallas as pl
from jax.experimental.pallas import tpu as pltpu
from jax.experimental.pallas import tpu_sc as plsc

_N_USERS = 5000
_N_ITEMS = 5000
_N = _N_USERS + _N_ITEMS
_D = 256
_DQ = 64
_NLAYERS = 4
_NE = 160000
_NS = 16
_NC = 2
_CB = 128
_NCH = 80
_ET = _NCH * _CB
_EP = _ET * _NS
_NP = 10240
_RPT = _NP // _NS
_ROW_CHUNKS = [(0, 128), (128, 128), (256, 128), (384, 128), (512, 128)]
_NBUF = 4


def _rsqrt_newton(x):
    i = lax.bitcast_convert_type(x, jnp.int32)
    i = jnp.int32(0x5F3759DF) - lax.shift_right_logical(i, 1)
    y = lax.bitcast_convert_type(i, jnp.float32)
    for _ in range(3):
        y = y * (jnp.float32(1.5) - jnp.float32(0.5) * x * y * y)
    return jnp.where(x > 0, y, jnp.float32(0.0))


def _sc_body(*refs):
    (eq0, eq1, eq2, eq3, row3, col3, val3) = refs[:7]
    lay = refs[7:23]
    fin = refs[23:27]
    (row_t, col_t, w_t, dis_t, gbuf0, gbuf1, gbuf2, gbuf3, zbuf, dtmp,
     sg0, sg1, sg2, sg3, ss0, ss1, ss2, ss3, sh_acc, sh_deg) = refs[7 + 20:]
    gbufs = [gbuf0, gbuf1, gbuf2, gbuf3]
    sem_g = [sg0, sg1, sg2, sg3]
    sem_s = [ss0, ss1, ss2, ss3]
    eq = [eq0, eq1, eq2, eq3]
    c = lax.axis_index("c")
    s = lax.axis_index("s")

    pltpu.sync_copy(row3.at[s], row_t)
    pltpu.sync_copy(col3.at[s], col_t)
    pltpu.sync_copy(val3.at[s], w_t)

    zv = jnp.zeros((16,), jnp.float32)
    for k in range(40):
        dtmp[pl.ds(16 * k, 16)] = zv

    def _zero_zbuf(i, _):
        for v in range(_DQ // 16):
            zbuf[i, pl.ds(16 * v, 16)] = zv
        return 0
    lax.fori_loop(0, _CB, _zero_zbuf, 0)

    pltpu.sync_copy(dtmp, sh_deg.at[pl.ds(s * 640, 640)])
    plsc.subcore_barrier()

    def _deg_fire(j, _):
        pltpu.async_copy(w_t.at[j], sh_deg.at[row_t.at[j]], ss0, add=True)
        return 0
    lax.fori_loop(0, _NCH, _deg_fire, 0)

    def _deg_drain(j, _):
        pltpu.make_async_copy(w_t.at[j], sh_deg.at[row_t.at[j]], ss0).wait()
        return 0
    lax.fori_loop(0, _NCH, _deg_drain, 0)
    plsc.subcore_barrier()

    pltpu.sync_copy(sh_deg.at[pl.ds(s * 640, 640)], dtmp)

    def _dis_vec(k, _):
        x = dtmp[pl.ds(16 * k, 16)]
        dtmp[pl.ds(16 * k, 16)] = _rsqrt_newton(x)
        return 0
    lax.fori_loop(0, 40, _dis_vec, 0)
    pltpu.sync_copy(dtmp, sh_deg.at[pl.ds(s * 640, 640)])
    plsc.subcore_barrier()
    pltpu.sync_copy(sh_deg, dis_t)

    def _wchunk(j, _):
        for i in range(8):
            sl = pl.ds(16 * i, 16)
            r16 = row_t[j, sl]
            c16 = col_t[j, sl]
            v16 = w_t[j, sl]
            a = plsc.load_gather(dis_t, [r16])
            b = plsc.load_gather(dis_t, [c16])
            w_t[j, sl] = a * v16 * b
        return 0
    lax.fori_loop(0, _NCH, _wchunk, 0)

    def _run(qq):
        srcs = [eq[qq]] + [lay[4 * l + qq] for l in range(_NLAYERS)]
        for off, sz in _ROW_CHUNKS:
            pltpu.sync_copy(zbuf.at[pl.ds(0, sz)],
                            sh_acc.at[pl.ds(s * _RPT + off, sz)])
        plsc.subcore_barrier()
        for layer in range(_NLAYERS):
            src = srcs[layer]
            dst = lay[4 * layer + qq]

            pltpu.async_copy(src.at[col_t.at[0]], gbufs[0], sem_g[0])
            pltpu.async_copy(src.at[col_t.at[1]], gbufs[1], sem_g[1])

            def _edge_quad(jj, _):
                for b in range(_NBUF):
                    j = _NBUF * jj + b
                    bn = (b + 2) % _NBUF
                    pltpu.make_async_copy(
                        src.at[col_t.at[j]], gbufs[b], sem_g[b]).wait()

                    @pl.when(j >= 2)
                    def _():
                        pltpu.make_async_copy(
                            gbufs[bn], sh_acc.at[row_t.at[j]],
                            sem_s[bn]).wait()

                    @pl.when(j + 2 < _NCH)
                    def _():
                        pltpu.async_copy(src.at[col_t.at[j + 2]],
                                         gbufs[bn], sem_g[bn])

                    def _scale(i, _, b=b):
                        wv = plsc.load_gather(
                            w_t, [jnp.full((16,), j, jnp.int32),
                                  jnp.full((16,), i, jnp.int32)])
                        for v in range(_DQ // 16):
                            sl = pl.ds(16 * v, 16)
                            gbufs[b][i, sl] = gbufs[b][i, sl] * wv
                        return 0
                    lax.fori_loop(0, _CB, _scale, 0)
                    pltpu.async_copy(gbufs[b], sh_acc.at[row_t.at[j]],
                                     sem_s[b], add=True)
                return 0
            lax.fori_loop(0, _NCH // _NBUF, _edge_quad, 0)
            for j in range(_NCH - 2, _NCH):
                pltpu.make_async_copy(
                    gbufs[j % _NBUF], sh_acc.at[row_t.at[j]],
                    sem_s[j % _NBUF]).wait()
            plsc.subcore_barrier()

            if layer < _NLAYERS - 1:
                for k, (off, sz) in enumerate(_ROW_CHUNKS):
                    kb = k % _NBUF
                    if k >= _NBUF:
                        pltpu.make_async_copy(
                            gbufs[kb],
                            dst.at[pl.ds(s * _RPT + _ROW_CHUNKS[k - _NBUF][0],
                                         sz)],
                            sem_s[kb]).wait()
                    pltpu.sync_copy(sh_acc.at[pl.ds(s * _RPT + off, sz)],
                                    gbufs[kb])
                    pltpu.sync_copy(zbuf,
                                    sh_acc.at[pl.ds(s * _RPT + off, sz)])
                    pltpu.async_copy(gbufs[kb],
                                     dst.at[pl.ds(s * _RPT + off, sz)],
                                     sem_s[kb])
                nch = len(_ROW_CHUNKS)
                for k in range(max(0, nch - _NBUF), nch):
                    kb = k % _NBUF
                    pltpu.make_async_copy(
                        gbufs[kb],
                        dst.at[pl.ds(s * _RPT + _ROW_CHUNKS[k][0], 128)],
                        sem_s[kb]).wait()
                plsc.subcore_barrier()

        fifth = jnp.float32(0.2)
        for off, sz in _ROW_CHUNKS:
            nb = s * _RPT + off
            pltpu.sync_copy(sh_acc.at[pl.ds(nb, sz)], zbuf)
            for t in range(4):
                pltpu.sync_copy(srcs[t].at[pl.ds(nb, sz)], gbuf0)

                def _acc(i, _):
                    for v in range(_DQ // 16):
                        sl = pl.ds(16 * v, 16)
                        zbuf[i, sl] = zbuf[i, sl] + gbuf0[i, sl]
                    return 0
                lax.fori_loop(0, sz, _acc, 0)

            def _fin(i, _):
                for v in range(_DQ // 16):
                    sl = pl.ds(16 * v, 16)
                    zbuf[i, sl] = zbuf[i, sl] * fifth
                return 0
            lax.fori_loop(0, sz, _fin, 0)
            pltpu.sync_copy(zbuf, fin[qq].at[pl.ds(nb, sz)])
        lax.fori_loop(0, _CB, _zero_zbuf, 0)

    @pl.when(c == 0)
    def _():
        _run(0)
        _run(1)

    @pl.when(c == 1)
    def _():
        _run(2)
        _run(3)


_quarter = jax.ShapeDtypeStruct((_NP, _DQ), jnp.float32)

_sc_kernel = functools.partial(
    pl.kernel,
    out_type=[_quarter] * 20,
    mesh=plsc.VectorSubcoreMesh(core_axis_name="c", subcore_axis_name="s",
                                num_cores=_NC, num_subcores=_NS),
    compiler_params=pltpu.CompilerParams(needs_layout_passes=False,
                                         use_tc_tiling_on_sc=False),
    scratch_types=[
        pltpu.VMEM((_NCH, _CB), jnp.int32),
        pltpu.VMEM((_NCH, _CB), jnp.int32),
        pltpu.VMEM((_NCH, _CB), jnp.float32),
        pltpu.VMEM((_NP,), jnp.float32),
        pltpu.VMEM((_CB, _DQ), jnp.float32),
        pltpu.VMEM((_CB, _DQ), jnp.float32),
        pltpu.VMEM((_CB, _DQ), jnp.float32),
        pltpu.VMEM((_CB, _DQ), jnp.float32),
        pltpu.VMEM((_CB, _DQ), jnp.float32),
        pltpu.VMEM((640,), jnp.float32),
        pltpu.SemaphoreType.DMA,
        pltpu.SemaphoreType.DMA,
        pltpu.SemaphoreType.DMA,
        pltpu.SemaphoreType.DMA,
        pltpu.SemaphoreType.DMA,
        pltpu.SemaphoreType.DMA,
        pltpu.SemaphoreType.DMA,
        pltpu.SemaphoreType.DMA,
        pltpu.VMEM_SHARED((_NP, _DQ), jnp.float32),
        pltpu.VMEM_SHARED((_NP,), jnp.float32),
    ],
)(_sc_body)


def kernel(values, E_u, E_v, edge_index):
    row = edge_index[0].astype(jnp.int32)
    col = edge_index[1].astype(jnp.int32)
    pad = _EP - _NE
    row3 = jnp.concatenate([row, jnp.zeros((pad,), jnp.int32)]).reshape(
        _NS, _NCH, _CB)
    col3 = jnp.concatenate([col, jnp.zeros((pad,), jnp.int32)]).reshape(
        _NS, _NCH, _CB)
    val3 = jnp.concatenate([values, jnp.zeros((pad,), values.dtype)]).reshape(
        _NS, _NCH, _CB)
    E_full = jnp.concatenate(
        [E_u, E_v, jnp.zeros((_NP - _N, _D), jnp.float32)], axis=0)
    eqs = [E_full[:, q * _DQ:(q + 1) * _DQ] for q in range(4)]
    outs = _sc_kernel(*eqs, row3, col3, val3)
    fin = outs[16:20]
    E_final = jnp.concatenate(fin, axis=1)
    return (E_final[:_N_USERS], E_final[_N_USERS:_N])
